# baseline (speedup 1.0000x reference)
import functools

import jax
import jax.numpy as jnp
from jax import lax
from jax.experimental import pallas as pl
from jax.experimental.pallas import tpu as pltpu
from jax.experimental.pallas import tpu_sc as plsc

N = 10000
E = 320000
D = 128
DH = D // 2

NSC = 2
NT = 16
NP = 10240
RPT = NP // NT
CH = 128
KCH = 160
BC = 8
KB = KCH // BC
EP = NT * KCH * CH

_mesh = plsc.VectorSubcoreMesh(core_axis_name="c", subcore_axis_name="s")


@functools.partial(
    pl.kernel,
    out_type=jax.ShapeDtypeStruct((NSC, NP), jnp.float32),
    mesh=_mesh,
    scratch_types=[
        pltpu.VMEM((BC, CH), jnp.int32),
        pltpu.VMEM((CH,), jnp.float32),
        pltpu.SemaphoreType.DMA,
        pltpu.VMEM_SHARED((NP,), jnp.float32),
    ],
)
def _deg_kernel(dst_hbm, ones_hbm, zeros_hbm, deg_out, idx_v, ones_v, sem, deg_sh):
    c = lax.axis_index("c")
    s = lax.axis_index("s")
    pltpu.sync_copy(ones_hbm, ones_v)
    pltpu.sync_copy(zeros_hbm, deg_sh.at[pl.ds(s * RPT, RPT)])
    plsc.subcore_barrier()

    def blk(ib, carry):
        pltpu.sync_copy(dst_hbm.at[c, s, pl.ds(ib * BC, BC)], idx_v)
        for j in range(BC):
            pltpu.sync_copy(ones_v, deg_sh.at[idx_v.at[j]], add=True)
        return carry

    lax.fori_loop(0, KB, blk, 0)
    plsc.subcore_barrier()
    pltpu.sync_copy(deg_sh.at[pl.ds(s * RPT, RPT)], deg_out.at[c, pl.ds(s * RPT, RPT)])


@functools.partial(
    pl.kernel,
    out_type=jax.ShapeDtypeStruct((NSC * NP, D), jnp.float32),
    mesh=_mesh,
    scratch_types=[
        pltpu.VMEM((BC, CH), jnp.int32),
        pltpu.VMEM((BC, CH), jnp.int32),
        pltpu.VMEM((CH, D), jnp.float32),
        pltpu.VMEM((CH, D), jnp.float32),
        pltpu.SemaphoreType.DMA,
        pltpu.SemaphoreType.DMA,
        pltpu.VMEM_SHARED((NP, D), jnp.float32),
    ],
)
def _agg_kernel(y_hbm, src_hbm, dst_hbm, zrow_hbm, agg_out,
                idxs_v, idxd_v, rb0, rb1, sg0, sg1, agg_sh):
    c = lax.axis_index("c")
    s = lax.axis_index("s")
    rbs = (rb0, rb1)
    sgs = (sg0, sg1)

    pltpu.sync_copy(zrow_hbm, agg_sh.at[pl.ds(s * RPT, RPT)])
    plsc.subcore_barrier()

    def blk(ib, carry):
        pltpu.sync_copy(src_hbm.at[c, s, pl.ds(ib * BC, BC)], idxs_v)
        pltpu.sync_copy(dst_hbm.at[c, s, pl.ds(ib * BC, BC)], idxd_v)
        pltpu.async_copy(y_hbm.at[idxs_v.at[0]], rb0, sg0)
        for j in range(BC):
            p = j % 2
            pltpu.make_async_copy(
                y_hbm.at[idxs_v.at[j]], rbs[p], sgs[p]).wait()
            if j + 1 < BC:
                pltpu.async_copy(
                    y_hbm.at[idxs_v.at[j + 1]], rbs[1 - p], sgs[1 - p])
            pltpu.sync_copy(rbs[p], agg_sh.at[idxd_v.at[j]], add=True)
        return carry

    lax.fori_loop(0, KB, blk, 0)
    plsc.subcore_barrier()
    pltpu.sync_copy(agg_sh.at[pl.ds(s * RPT, RPT)],
                    agg_out.at[pl.ds(c * NP + s * RPT, RPT)])


_BM = 256


def _prep_body(x_ref, deg_ref, w_ref, y_ref):
    dis = lax.rsqrt(deg_ref[...] + 1.0)
    xw = jnp.dot(x_ref[...], w_ref[...], preferred_element_type=jnp.float32)
    y_ref[...] = xw * dis


def _prep_call(xcat, deg2d, W):
    grid = (NSC * NP) // _BM
    return pl.pallas_call(
        _prep_body,
        grid=(grid,),
        in_specs=[
            pl.BlockSpec((_BM, D), lambda i: (i, 0)),
            pl.BlockSpec((_BM, 1), lambda i: (i, 0)),
            pl.BlockSpec((D, D), lambda i: (0, 0)),
        ],
        out_specs=pl.BlockSpec((_BM, D), lambda i: (i, 0)),
        out_shape=jax.ShapeDtypeStruct((NSC * NP, D), jnp.float32),
    )(xcat, deg2d, W)


_FM = 80
_G2 = NP // _FM


def _finish_body(a1_ref, y1_ref, d1_ref, a2_ref, y2_ref, d2_ref, b_ref,
                 h1_ref, h2_ref):
    b = b_ref[...]
    dis1 = lax.rsqrt(d1_ref[...] + 1.0)
    dis2 = lax.rsqrt(d2_ref[...] + 1.0)
    h1_ref[...] = jnp.maximum(dis1 * (a1_ref[...] + y1_ref[...]) + b, 0.0)
    h2_ref[...] = jnp.maximum(dis2 * (a2_ref[...] + y2_ref[...]) + b, 0.0)


def _finish_call(agg, y, deg2d, b2d):
    grid = N // _FM
    g1d = pl.BlockSpec((_FM, D), lambda i: (i, 0))
    g1s = pl.BlockSpec((_FM, 1), lambda i: (i, 0))
    g2d = pl.BlockSpec((_FM, D), lambda i: (i + _G2, 0))
    g2s = pl.BlockSpec((_FM, 1), lambda i: (i + _G2, 0))
    out = pl.BlockSpec((_FM, D), lambda i: (i, 0))
    return pl.pallas_call(
        _finish_body,
        grid=(grid,),
        in_specs=[g1d, g1d, g1s, g2d, g2d, g2s,
                  pl.BlockSpec((1, D), lambda i: (0, 0))],
        out_specs=[out, out],
        out_shape=[jax.ShapeDtypeStruct((N, D), jnp.float32),
                   jax.ShapeDtypeStruct((N, D), jnp.float32)],
    )(agg, y, deg2d, agg, y, deg2d, b2d)


def kernel(x1, edge_index1, x2, edge_index2, W, b):
    padlen = EP - E
    e1 = edge_index1.astype(jnp.int32)
    e2 = edge_index2.astype(jnp.int32)
    src = jnp.concatenate(
        [e1[0], jnp.full((padlen,), N, jnp.int32),
         e2[0] + NP, jnp.full((padlen,), NP + N, jnp.int32)]
    ).reshape(NSC, NT, KCH, CH)
    dst = jnp.concatenate(
        [e1[1], jnp.full((padlen,), N, jnp.int32),
         e2[1], jnp.full((padlen,), N, jnp.int32)]
    ).reshape(NSC, NT, KCH, CH)
    zpad = jnp.zeros((NP - N, D), jnp.float32)
    xcat = jnp.concatenate([x1, zpad, x2, zpad])

    ones_ch = jnp.ones((CH,), jnp.float32)
    zeros_r = jnp.zeros((RPT,), jnp.float32)
    zeros_rd = jnp.zeros((RPT, D), jnp.float32)

    deg2d = _deg_kernel(dst, ones_ch, zeros_r).reshape(NSC * NP, 1)
    y = _prep_call(xcat, deg2d, W)
    agg = _agg_kernel(y, src, dst, zeros_rd)
    h1, h2 = _finish_call(agg, y, deg2d, b.reshape(1, D))
    return h1, h2


# speedup vs baseline: 1.0769x; 1.0769x over previous
import functools

import jax
import jax.numpy as jnp
from jax import lax
from jax.experimental import pallas as pl
from jax.experimental.pallas import tpu as pltpu
from jax.experimental.pallas import tpu_sc as plsc

N = 10000
E = 320000
D = 128
DH = D // 2

NSC = 2
NT = 16
NP = 10240
RPT = NP // NT
CH = 128
KCH = 160
BC = 8
KB = KCH // BC
EP = NT * KCH * CH

_mesh = plsc.VectorSubcoreMesh(core_axis_name="c", subcore_axis_name="s")


@functools.partial(
    pl.kernel,
    out_type=jax.ShapeDtypeStruct((NSC, NP), jnp.float32),
    mesh=_mesh,
    scratch_types=[
        pltpu.VMEM((BC, CH), jnp.int32),
        pltpu.VMEM((CH,), jnp.float32),
        pltpu.SemaphoreType.DMA,
        pltpu.VMEM_SHARED((NP,), jnp.float32),
    ],
)
def _deg_kernel(dst_hbm, ones_hbm, zeros_hbm, deg_out, idx_v, ones_v, sem, deg_sh):
    c = lax.axis_index("c")
    s = lax.axis_index("s")
    pltpu.sync_copy(ones_hbm, ones_v)
    pltpu.sync_copy(zeros_hbm, deg_sh.at[pl.ds(s * RPT, RPT)])
    plsc.subcore_barrier()

    def blk(ib, carry):
        pltpu.sync_copy(dst_hbm.at[c, s, pl.ds(ib * BC, BC)], idx_v)
        for j in range(BC):
            pltpu.sync_copy(ones_v, deg_sh.at[idx_v.at[j]], add=True)
        return carry

    lax.fori_loop(0, KB, blk, 0)
    plsc.subcore_barrier()
    pltpu.sync_copy(deg_sh.at[pl.ds(s * RPT, RPT)], deg_out.at[c, pl.ds(s * RPT, RPT)])


@functools.partial(
    pl.kernel,
    out_type=jax.ShapeDtypeStruct((NSC * NP, D), jnp.float32),
    mesh=_mesh,
    scratch_types=[
        pltpu.VMEM((BC, CH), jnp.int32),
        pltpu.VMEM((BC, CH), jnp.int32),
        pltpu.VMEM((CH, D), jnp.float32),
        pltpu.VMEM((CH, D), jnp.float32),
        pltpu.SemaphoreType.DMA,
        pltpu.SemaphoreType.DMA,
        pltpu.VMEM_SHARED((NP, D), jnp.float32),
    ],
)
def _agg_kernel(y_hbm, src_hbm, dst_hbm, zrow_hbm, agg_out,
                idxs_v, idxd_v, rb0, rb1, sg0, sg1, agg_sh):
    c = lax.axis_index("c")
    s = lax.axis_index("s")
    rbs = (rb0, rb1)
    sgs = (sg0, sg1)

    pltpu.sync_copy(zrow_hbm, agg_sh.at[pl.ds(s * RPT, RPT)])
    plsc.subcore_barrier()

    def blk(ib, carry):
        pltpu.sync_copy(src_hbm.at[c, s, pl.ds(ib * BC, BC)], idxs_v)
        pltpu.sync_copy(dst_hbm.at[c, s, pl.ds(ib * BC, BC)], idxd_v)
        pltpu.async_copy(y_hbm.at[idxs_v.at[0]], rb0, sg0)
        for j in range(BC):
            p = j % 2
            pltpu.make_async_copy(
                y_hbm.at[idxs_v.at[j]], rbs[p], sgs[p]).wait()
            if j + 1 < BC:
                pltpu.async_copy(
                    y_hbm.at[idxs_v.at[j + 1]], rbs[1 - p], sgs[1 - p])
            pltpu.sync_copy(rbs[p], agg_sh.at[idxd_v.at[j]], add=True)
        return carry

    lax.fori_loop(0, KB, blk, 0)
    plsc.subcore_barrier()
    pltpu.sync_copy(agg_sh.at[pl.ds(s * RPT, RPT)],
                    agg_out.at[pl.ds(c * NP + s * RPT, RPT)])


_BM = 256


def _prep_body(x_ref, deg_ref, w_ref, y_ref, dis_ref):
    dis = lax.rsqrt(deg_ref[...] + 1.0)
    xw = jnp.dot(x_ref[...], w_ref[...], preferred_element_type=jnp.float32)
    y_ref[...] = xw * dis
    dis_ref[...] = dis


def _prep_call(xcat, deg2d, W):
    grid = (NSC * NP) // _BM
    return pl.pallas_call(
        _prep_body,
        grid=(grid,),
        in_specs=[
            pl.BlockSpec((_BM, D), lambda i: (i, 0)),
            pl.BlockSpec((_BM, 1), lambda i: (i, 0)),
            pl.BlockSpec((D, D), lambda i: (0, 0)),
        ],
        out_specs=[
            pl.BlockSpec((_BM, D), lambda i: (i, 0)),
            pl.BlockSpec((_BM, 1), lambda i: (i, 0)),
        ],
        out_shape=[
            jax.ShapeDtypeStruct((NSC * NP, D), jnp.float32),
            jax.ShapeDtypeStruct((NSC * NP, 1), jnp.float32),
        ],
    )(xcat, deg2d, W)


def _finish_body(agg_ref, y_ref, dis_ref, b_ref, out_ref):
    out_ref[...] = jnp.maximum(
        dis_ref[...] * (agg_ref[...] + y_ref[...]) + b_ref[...], 0.0)


def _finish_call(agg, y, dis, b2d):
    grid = (NSC * NP) // _BM
    return pl.pallas_call(
        _finish_body,
        grid=(grid,),
        in_specs=[
            pl.BlockSpec((_BM, D), lambda i: (i, 0)),
            pl.BlockSpec((_BM, D), lambda i: (i, 0)),
            pl.BlockSpec((_BM, 1), lambda i: (i, 0)),
            pl.BlockSpec((1, D), lambda i: (0, 0)),
        ],
        out_specs=pl.BlockSpec((_BM, D), lambda i: (i, 0)),
        out_shape=jax.ShapeDtypeStruct((NSC * NP, D), jnp.float32),
    )(agg, y, dis, b2d)


def _prep_edges(edge_index, g):
    src = edge_index[0].astype(jnp.int32)
    dst = edge_index[1].astype(jnp.int32)
    padlen = EP - E
    src = jnp.concatenate(
        [src + g * NP, jnp.full((padlen,), g * NP + N, jnp.int32)])
    dst = jnp.concatenate([dst, jnp.full((padlen,), N, jnp.int32)])
    return src.reshape(NT, KCH, CH), dst.reshape(NT, KCH, CH)


def kernel(x1, edge_index1, x2, edge_index2, W, b):
    s1, d1 = _prep_edges(edge_index1, 0)
    s2, d2 = _prep_edges(edge_index2, 1)
    src = jnp.stack([s1, s2])
    dst = jnp.stack([d1, d2])
    zpad = jnp.zeros((NP - N, D), jnp.float32)
    xcat = jnp.concatenate([x1, zpad, x2, zpad])

    ones_ch = jnp.ones((CH,), jnp.float32)
    zeros_r = jnp.zeros((RPT,), jnp.float32)
    zeros_rd = jnp.zeros((RPT, D), jnp.float32)

    deg = _deg_kernel(dst, ones_ch, zeros_r)
    y, dis = _prep_call(xcat, deg.reshape(NSC * NP, 1), W)
    agg = _agg_kernel(y, src, dst, zeros_rd)
    out = _finish_call(agg, y, dis, b.reshape(1, D))
    return out[:N], out[NP:NP + N]


# speedup vs baseline: 1.1002x; 1.0216x over previous
import functools

import jax
import jax.numpy as jnp
from jax import lax
from jax.experimental import pallas as pl
from jax.experimental.pallas import tpu as pltpu
from jax.experimental.pallas import tpu_sc as plsc

N = 10000
E = 320000
D = 128
DH = D // 2

NSC = 2
NT = 16
NP = 10240
RPT = NP // NT
CH = 128
KCH = 160
BC = 8
KB = KCH // BC
EP = NT * KCH * CH

_mesh = plsc.VectorSubcoreMesh(core_axis_name="c", subcore_axis_name="s")


@functools.partial(
    pl.kernel,
    out_type=jax.ShapeDtypeStruct((NSC, NP), jnp.float32),
    mesh=_mesh,
    scratch_types=[
        pltpu.VMEM((BC, CH), jnp.int32),
        pltpu.VMEM((CH,), jnp.float32),
        pltpu.SemaphoreType.DMA,
        pltpu.VMEM_SHARED((NP,), jnp.float32),
    ],
)
def _deg_kernel(dst_hbm, ones_hbm, zeros_hbm, deg_out, idx_v, ones_v, sem, deg_sh):
    c = lax.axis_index("c")
    s = lax.axis_index("s")
    pltpu.sync_copy(ones_hbm, ones_v)
    pltpu.sync_copy(zeros_hbm, deg_sh.at[pl.ds(s * RPT, RPT)])
    plsc.subcore_barrier()

    def blk(ib, carry):
        pltpu.sync_copy(dst_hbm.at[c, s, pl.ds(ib * BC, BC)], idx_v)
        for j in range(BC):
            pltpu.sync_copy(ones_v, deg_sh.at[idx_v.at[j]], add=True)
        return carry

    lax.fori_loop(0, KB, blk, 0)
    plsc.subcore_barrier()
    pltpu.sync_copy(deg_sh.at[pl.ds(s * RPT, RPT)], deg_out.at[c, pl.ds(s * RPT, RPT)])


@functools.partial(
    pl.kernel,
    out_type=jax.ShapeDtypeStruct((NSC * NP, D), jnp.float32),
    mesh=_mesh,
    scratch_types=[
        pltpu.VMEM((2, BC, CH), jnp.int32),
        pltpu.VMEM((2, BC, CH), jnp.int32),
        pltpu.VMEM((CH, D), jnp.float32),
        pltpu.VMEM((CH, D), jnp.float32),
        pltpu.SemaphoreType.DMA,
        pltpu.SemaphoreType.DMA,
        pltpu.SemaphoreType.DMA,
        pltpu.VMEM_SHARED((NP, D), jnp.float32),
    ],
)
def _agg_kernel(y_hbm, src_hbm, dst_hbm, zrow_hbm, agg_out,
                idxs_v, idxd_v, rb0, rb1, sg0, sg1, si, agg_sh):
    c = lax.axis_index("c")
    s = lax.axis_index("s")
    rbs = (rb0, rb1)
    sgs = (sg0, sg1)

    pltpu.async_copy(src_hbm.at[c, s, pl.ds(0, BC)], idxs_v.at[0], si)
    pltpu.async_copy(dst_hbm.at[c, s, pl.ds(0, BC)], idxd_v.at[0], si)
    pltpu.sync_copy(zrow_hbm, agg_sh.at[pl.ds(s * RPT, RPT)])
    plsc.subcore_barrier()

    def blk(ib, carry):
        par = lax.rem(ib, 2)
        pltpu.make_async_copy(
            src_hbm.at[c, s, pl.ds(ib * BC, BC)], idxs_v.at[par], si).wait()
        pltpu.make_async_copy(
            dst_hbm.at[c, s, pl.ds(ib * BC, BC)], idxd_v.at[par], si).wait()

        @pl.when(ib + 1 < KB)
        def _prefetch():
            pltpu.async_copy(src_hbm.at[c, s, pl.ds((ib + 1) * BC, BC)],
                             idxs_v.at[1 - par], si)
            pltpu.async_copy(dst_hbm.at[c, s, pl.ds((ib + 1) * BC, BC)],
                             idxd_v.at[1 - par], si)

        pltpu.async_copy(y_hbm.at[idxs_v.at[par, 0]], rb0, sg0)
        for j in range(BC):
            p = j % 2
            pltpu.make_async_copy(
                y_hbm.at[idxs_v.at[par, j]], rbs[p], sgs[p]).wait()
            if j + 1 < BC:
                pltpu.async_copy(
                    y_hbm.at[idxs_v.at[par, j + 1]], rbs[1 - p], sgs[1 - p])
            pltpu.sync_copy(rbs[p], agg_sh.at[idxd_v.at[par, j]], add=True)
        return carry

    lax.fori_loop(0, KB, blk, 0)
    plsc.subcore_barrier()
    pltpu.sync_copy(agg_sh.at[pl.ds(s * RPT, RPT)],
                    agg_out.at[pl.ds(c * NP + s * RPT, RPT)])


_BM = 256


def _prep_body(x_ref, deg_ref, w_ref, y_ref, dis_ref):
    dis = lax.rsqrt(deg_ref[...] + 1.0)
    xw = jnp.dot(x_ref[...], w_ref[...], preferred_element_type=jnp.float32)
    y_ref[...] = xw * dis
    dis_ref[...] = dis


def _prep_call(xcat, deg2d, W):
    grid = (NSC * NP) // _BM
    return pl.pallas_call(
        _prep_body,
        grid=(grid,),
        in_specs=[
            pl.BlockSpec((_BM, D), lambda i: (i, 0)),
            pl.BlockSpec((_BM, 1), lambda i: (i, 0)),
            pl.BlockSpec((D, D), lambda i: (0, 0)),
        ],
        out_specs=[
            pl.BlockSpec((_BM, D), lambda i: (i, 0)),
            pl.BlockSpec((_BM, 1), lambda i: (i, 0)),
        ],
        out_shape=[
            jax.ShapeDtypeStruct((NSC * NP, D), jnp.float32),
            jax.ShapeDtypeStruct((NSC * NP, 1), jnp.float32),
        ],
    )(xcat, deg2d, W)


def _finish_body(agg_ref, y_ref, dis_ref, b_ref, out_ref):
    out_ref[...] = jnp.maximum(
        dis_ref[...] * (agg_ref[...] + y_ref[...]) + b_ref[...], 0.0)


def _finish_call(agg, y, dis, b2d):
    grid = (NSC * NP) // _BM
    return pl.pallas_call(
        _finish_body,
        grid=(grid,),
        in_specs=[
            pl.BlockSpec((_BM, D), lambda i: (i, 0)),
            pl.BlockSpec((_BM, D), lambda i: (i, 0)),
            pl.BlockSpec((_BM, 1), lambda i: (i, 0)),
            pl.BlockSpec((1, D), lambda i: (0, 0)),
        ],
        out_specs=pl.BlockSpec((_BM, D), lambda i: (i, 0)),
        out_shape=jax.ShapeDtypeStruct((NSC * NP, D), jnp.float32),
    )(agg, y, dis, b2d)


def _prep_edges(edge_index, g):
    src = edge_index[0].astype(jnp.int32)
    dst = edge_index[1].astype(jnp.int32)
    padlen = EP - E
    src = jnp.concatenate(
        [src + g * NP, jnp.full((padlen,), g * NP + N, jnp.int32)])
    dst = jnp.concatenate([dst, jnp.full((padlen,), N, jnp.int32)])
    return src.reshape(NT, KCH, CH), dst.reshape(NT, KCH, CH)


def kernel(x1, edge_index1, x2, edge_index2, W, b):
    s1, d1 = _prep_edges(edge_index1, 0)
    s2, d2 = _prep_edges(edge_index2, 1)
    src = jnp.stack([s1, s2])
    dst = jnp.stack([d1, d2])
    zpad = jnp.zeros((NP - N, D), jnp.float32)
    xcat = jnp.concatenate([x1, zpad, x2, zpad])

    ones_ch = jnp.ones((CH,), jnp.float32)
    zeros_r = jnp.zeros((RPT,), jnp.float32)
    zeros_rd = jnp.zeros((RPT, D), jnp.float32)

    deg = _deg_kernel(dst, ones_ch, zeros_r)
    y, dis = _prep_call(xcat, deg.reshape(NSC * NP, 1), W)
    agg = _agg_kernel(y, src, dst, zeros_rd)
    out = _finish_call(agg, y, dis, b.reshape(1, D))
    return out[:N], out[NP:NP + N]


# speedup vs baseline: 1.1009x; 1.0006x over previous
import functools

import jax
import jax.numpy as jnp
from jax import lax
from jax.experimental import pallas as pl
from jax.experimental.pallas import tpu as pltpu
from jax.experimental.pallas import tpu_sc as plsc

N = 10000
E = 320000
D = 128
DH = D // 2

NSC = 2
NT = 16
NP = 10240
RPT = NP // NT
CH = 128
KCH = 160
BC = 8
KB = KCH // BC
EP = NT * KCH * CH

_mesh = plsc.VectorSubcoreMesh(core_axis_name="c", subcore_axis_name="s")


@functools.partial(
    pl.kernel,
    out_type=jax.ShapeDtypeStruct((NSC, NP), jnp.float32),
    mesh=_mesh,
    scratch_types=[
        pltpu.VMEM((BC, CH), jnp.int32),
        pltpu.VMEM((CH,), jnp.float32),
        pltpu.SemaphoreType.DMA,
        pltpu.VMEM_SHARED((NP,), jnp.float32),
    ],
)
def _deg_kernel(dst_hbm, ones_hbm, zeros_hbm, deg_out, idx_v, ones_v, sem, deg_sh):
    c = lax.axis_index("c")
    s = lax.axis_index("s")
    pltpu.sync_copy(ones_hbm, ones_v)
    pltpu.sync_copy(zeros_hbm, deg_sh.at[pl.ds(s * RPT, RPT)])
    plsc.subcore_barrier()

    def blk(ib, carry):
        pltpu.sync_copy(dst_hbm.at[c, s, pl.ds(ib * BC, BC)], idx_v)
        for j in range(BC):
            pltpu.sync_copy(ones_v, deg_sh.at[idx_v.at[j]], add=True)
        return carry

    lax.fori_loop(0, KB, blk, 0)
    plsc.subcore_barrier()
    pltpu.sync_copy(deg_sh.at[pl.ds(s * RPT, RPT)], deg_out.at[c, pl.ds(s * RPT, RPT)])


@functools.partial(
    pl.kernel,
    out_type=jax.ShapeDtypeStruct((NSC * NP, D), jnp.float32),
    mesh=_mesh,
    scratch_types=[
        pltpu.VMEM((2, BC, CH), jnp.int32),
        pltpu.VMEM((2, BC, CH), jnp.int32),
        pltpu.VMEM((CH, D), jnp.float32),
        pltpu.VMEM((CH, D), jnp.float32),
        pltpu.SemaphoreType.DMA,
        pltpu.SemaphoreType.DMA,
        pltpu.SemaphoreType.DMA,
        pltpu.VMEM_SHARED((NP, D), jnp.float32),
    ],
)
def _agg_kernel(y_hbm, src_hbm, dst_hbm, zrow_hbm, agg_out,
                idxs_v, idxd_v, rb0, rb1, sg0, sg1, si, agg_sh):
    c = lax.axis_index("c")
    s = lax.axis_index("s")
    rbs = (rb0, rb1)
    sgs = (sg0, sg1)

    pltpu.sync_copy(src_hbm.at[c, s, pl.ds(0, BC)], idxs_v.at[0])
    pltpu.sync_copy(dst_hbm.at[c, s, pl.ds(0, BC)], idxd_v.at[0])
    pltpu.async_copy(src_hbm.at[c, s, pl.ds(BC, BC)], idxs_v.at[1], si)
    pltpu.async_copy(dst_hbm.at[c, s, pl.ds(BC, BC)], idxd_v.at[1], si)
    pltpu.async_copy(y_hbm.at[idxs_v.at[0, 0]], rb0, sg0)
    pltpu.sync_copy(zrow_hbm, agg_sh.at[pl.ds(s * RPT, RPT)])
    plsc.subcore_barrier()

    def blk(ib, carry):
        par = lax.rem(ib, 2)
        for j in range(BC):
            p = j % 2
            pltpu.make_async_copy(
                y_hbm.at[idxs_v.at[par, j]], rbs[p], sgs[p]).wait()
            if j + 1 < BC:
                pltpu.async_copy(
                    y_hbm.at[idxs_v.at[par, j + 1]], rbs[1 - p], sgs[1 - p])
            pltpu.sync_copy(rbs[p], agg_sh.at[idxd_v.at[par, j]], add=True)

        @pl.when(ib + 1 < KB)
        def _next_block():
            pltpu.make_async_copy(
                src_hbm.at[c, s, pl.ds((ib + 1) * BC, BC)],
                idxs_v.at[1 - par], si).wait()
            pltpu.make_async_copy(
                dst_hbm.at[c, s, pl.ds((ib + 1) * BC, BC)],
                idxd_v.at[1 - par], si).wait()

            @pl.when(ib + 2 < KB)
            def _prefetch():
                pltpu.async_copy(src_hbm.at[c, s, pl.ds((ib + 2) * BC, BC)],
                                 idxs_v.at[par], si)
                pltpu.async_copy(dst_hbm.at[c, s, pl.ds((ib + 2) * BC, BC)],
                                 idxd_v.at[par], si)

            pltpu.async_copy(y_hbm.at[idxs_v.at[1 - par, 0]], rb0, sg0)
        return carry

    lax.fori_loop(0, KB, blk, 0)
    plsc.subcore_barrier()
    pltpu.sync_copy(agg_sh.at[pl.ds(s * RPT, RPT)],
                    agg_out.at[pl.ds(c * NP + s * RPT, RPT)])


_BM = 256


def _prep_body(x_ref, deg_ref, w_ref, y_ref, dis_ref):
    dis = lax.rsqrt(deg_ref[...] + 1.0)
    xw = jnp.dot(x_ref[...], w_ref[...], preferred_element_type=jnp.float32)
    y_ref[...] = xw * dis
    dis_ref[...] = dis


def _prep_call(xcat, deg2d, W):
    grid = (NSC * NP) // _BM
    return pl.pallas_call(
        _prep_body,
        grid=(grid,),
        in_specs=[
            pl.BlockSpec((_BM, D), lambda i: (i, 0)),
            pl.BlockSpec((_BM, 1), lambda i: (i, 0)),
            pl.BlockSpec((D, D), lambda i: (0, 0)),
        ],
        out_specs=[
            pl.BlockSpec((_BM, D), lambda i: (i, 0)),
            pl.BlockSpec((_BM, 1), lambda i: (i, 0)),
        ],
        out_shape=[
            jax.ShapeDtypeStruct((NSC * NP, D), jnp.float32),
            jax.ShapeDtypeStruct((NSC * NP, 1), jnp.float32),
        ],
    )(xcat, deg2d, W)


def _finish_body(agg_ref, y_ref, dis_ref, b_ref, out_ref):
    out_ref[...] = jnp.maximum(
        dis_ref[...] * (agg_ref[...] + y_ref[...]) + b_ref[...], 0.0)


def _finish_call(agg, y, dis, b2d):
    grid = (NSC * NP) // _BM
    return pl.pallas_call(
        _finish_body,
        grid=(grid,),
        in_specs=[
            pl.BlockSpec((_BM, D), lambda i: (i, 0)),
            pl.BlockSpec((_BM, D), lambda i: (i, 0)),
            pl.BlockSpec((_BM, 1), lambda i: (i, 0)),
            pl.BlockSpec((1, D), lambda i: (0, 0)),
        ],
        out_specs=pl.BlockSpec((_BM, D), lambda i: (i, 0)),
        out_shape=jax.ShapeDtypeStruct((NSC * NP, D), jnp.float32),
    )(agg, y, dis, b2d)


def _prep_edges(edge_index, g):
    src = edge_index[0].astype(jnp.int32)
    dst = edge_index[1].astype(jnp.int32)
    padlen = EP - E
    src = jnp.concatenate(
        [src + g * NP, jnp.full((padlen,), g * NP + N, jnp.int32)])
    dst = jnp.concatenate([dst, jnp.full((padlen,), N, jnp.int32)])
    return src.reshape(NT, KCH, CH), dst.reshape(NT, KCH, CH)


def kernel(x1, edge_index1, x2, edge_index2, W, b):
    s1, d1 = _prep_edges(edge_index1, 0)
    s2, d2 = _prep_edges(edge_index2, 1)
    src = jnp.stack([s1, s2])
    dst = jnp.stack([d1, d2])
    zpad = jnp.zeros((NP - N, D), jnp.float32)
    xcat = jnp.concatenate([x1, zpad, x2, zpad])

    ones_ch = jnp.ones((CH,), jnp.float32)
    zeros_r = jnp.zeros((RPT,), jnp.float32)
    zeros_rd = jnp.zeros((RPT, D), jnp.float32)

    deg = _deg_kernel(dst, ones_ch, zeros_r)
    y, dis = _prep_call(xcat, deg.reshape(NSC * NP, 1), W)
    agg = _agg_kernel(y, src, dst, zeros_rd)
    out = _finish_call(agg, y, dis, b.reshape(1, D))
    return out[:N], out[NP:NP + N]


# speedup vs baseline: 1.1009x; 1.0000x over previous
import functools

import jax
import jax.numpy as jnp
from jax import lax
from jax.experimental import pallas as pl
from jax.experimental.pallas import tpu as pltpu
from jax.experimental.pallas import tpu_sc as plsc

N = 10000
E = 320000
D = 128

NSC = 2
NT = 16
NP = 10240
RPT = NP // NT
CH = 128
KCH = 160
BC = 8
KB = KCH // BC
EP = NT * KCH * CH

_mesh = plsc.VectorSubcoreMesh(core_axis_name="c", subcore_axis_name="s")


@functools.partial(
    pl.kernel,
    out_type=jax.ShapeDtypeStruct((NSC, NP), jnp.float32),
    mesh=_mesh,
    scratch_types=[
        pltpu.VMEM((BC, CH), jnp.int32),
        pltpu.VMEM((CH,), jnp.float32),
        pltpu.SemaphoreType.DMA,
        pltpu.VMEM_SHARED((NP,), jnp.float32),
    ],
)
def _deg_kernel(dst_hbm, ones_hbm, zeros_hbm, deg_out, idx_v, ones_v, sem, deg_sh):
    c = lax.axis_index("c")
    s = lax.axis_index("s")
    pltpu.sync_copy(ones_hbm, ones_v)
    pltpu.sync_copy(zeros_hbm, deg_sh.at[pl.ds(s * RPT, RPT)])
    plsc.subcore_barrier()

    def blk(ib, carry):
        pltpu.sync_copy(dst_hbm.at[c, s, pl.ds(ib * BC, BC)], idx_v)
        for j in range(BC):
            pltpu.sync_copy(ones_v, deg_sh.at[idx_v.at[j]], add=True)
        return carry

    lax.fori_loop(0, KB, blk, 0)
    plsc.subcore_barrier()
    pltpu.sync_copy(deg_sh.at[pl.ds(s * RPT, RPT)], deg_out.at[c, pl.ds(s * RPT, RPT)])


@functools.partial(
    pl.kernel,
    out_type=jax.ShapeDtypeStruct((NSC * NP, D), jnp.float32),
    mesh=_mesh,
    scratch_types=[
        pltpu.VMEM((2, BC, CH), jnp.int32),
        pltpu.VMEM((2, BC, CH), jnp.int32),
        pltpu.VMEM((CH, D), jnp.float32),
        pltpu.VMEM((CH, D), jnp.float32),
        pltpu.SemaphoreType.DMA,
        pltpu.SemaphoreType.DMA,
        pltpu.SemaphoreType.DMA,
        pltpu.VMEM_SHARED((NP, D), jnp.float32),
    ],
)
def _agg_kernel(y_hbm, src_hbm, dst_hbm, zrow_hbm, agg_out,
                idxs_v, idxd_v, rb0, rb1, sg0, sg1, si, agg_sh):
    c = lax.axis_index("c")
    s = lax.axis_index("s")
    rbs = (rb0, rb1)
    sgs = (sg0, sg1)

    pltpu.sync_copy(src_hbm.at[c, s, pl.ds(0, BC)], idxs_v.at[0])
    pltpu.sync_copy(dst_hbm.at[c, s, pl.ds(0, BC)], idxd_v.at[0])
    pltpu.async_copy(src_hbm.at[c, s, pl.ds(BC, BC)], idxs_v.at[1], si)
    pltpu.async_copy(dst_hbm.at[c, s, pl.ds(BC, BC)], idxd_v.at[1], si)
    pltpu.async_copy(y_hbm.at[idxs_v.at[0, 0]], rb0, sg0)
    pltpu.sync_copy(zrow_hbm, agg_sh.at[pl.ds(s * RPT, RPT)])
    plsc.subcore_barrier()

    def blk(ib, carry):
        par = lax.rem(ib, 2)
        for j in range(BC):
            p = j % 2
            pltpu.make_async_copy(
                y_hbm.at[idxs_v.at[par, j]], rbs[p], sgs[p]).wait()
            if j + 1 < BC:
                pltpu.async_copy(
                    y_hbm.at[idxs_v.at[par, j + 1]], rbs[1 - p], sgs[1 - p])
            pltpu.sync_copy(rbs[p], agg_sh.at[idxd_v.at[par, j]], add=True)

        @pl.when(ib + 1 < KB)
        def _next_block():
            pltpu.make_async_copy(
                src_hbm.at[c, s, pl.ds((ib + 1) * BC, BC)],
                idxs_v.at[1 - par], si).wait()
            pltpu.make_async_copy(
                dst_hbm.at[c, s, pl.ds((ib + 1) * BC, BC)],
                idxd_v.at[1 - par], si).wait()

            @pl.when(ib + 2 < KB)
            def _prefetch():
                pltpu.async_copy(src_hbm.at[c, s, pl.ds((ib + 2) * BC, BC)],
                                 idxs_v.at[par], si)
                pltpu.async_copy(dst_hbm.at[c, s, pl.ds((ib + 2) * BC, BC)],
                                 idxd_v.at[par], si)

            pltpu.async_copy(y_hbm.at[idxs_v.at[1 - par, 0]], rb0, sg0)
        return carry

    lax.fori_loop(0, KB, blk, 0)
    plsc.subcore_barrier()
    pltpu.sync_copy(agg_sh.at[pl.ds(s * RPT, RPT)],
                    agg_out.at[pl.ds(c * NP + s * RPT, RPT)])


_BM = 256


def _prep_body(x_ref, deg_ref, w_ref, y_ref, dis_ref):
    dis = lax.rsqrt(deg_ref[...] + 1.0)
    xw = jnp.dot(x_ref[...], w_ref[...], preferred_element_type=jnp.float32)
    y_ref[...] = xw * dis
    dis_ref[...] = dis


def _prep_call(xcat, deg2d, W):
    grid = (NSC * NP) // _BM
    return pl.pallas_call(
        _prep_body,
        grid=(grid,),
        in_specs=[
            pl.BlockSpec((_BM, D), lambda i: (i, 0)),
            pl.BlockSpec((_BM, 1), lambda i: (i, 0)),
            pl.BlockSpec((D, D), lambda i: (0, 0)),
        ],
        out_specs=[
            pl.BlockSpec((_BM, D), lambda i: (i, 0)),
            pl.BlockSpec((_BM, 1), lambda i: (i, 0)),
        ],
        out_shape=[
            jax.ShapeDtypeStruct((NSC * NP, D), jnp.float32),
            jax.ShapeDtypeStruct((NSC * NP, 1), jnp.float32),
        ],
    )(xcat, deg2d, W)


def _finish_body(agg_ref, y_ref, dis_ref, b_ref, out_ref):
    out_ref[...] = jnp.maximum(
        dis_ref[...] * (agg_ref[...] + y_ref[...]) + b_ref[...], 0.0)


def _finish_call(agg, y, dis, b2d):
    grid = (NSC * NP) // _BM
    return pl.pallas_call(
        _finish_body,
        grid=(grid,),
        in_specs=[
            pl.BlockSpec((_BM, D), lambda i: (i, 0)),
            pl.BlockSpec((_BM, D), lambda i: (i, 0)),
            pl.BlockSpec((_BM, 1), lambda i: (i, 0)),
            pl.BlockSpec((1, D), lambda i: (0, 0)),
        ],
        out_specs=pl.BlockSpec((_BM, D), lambda i: (i, 0)),
        out_shape=jax.ShapeDtypeStruct((NSC * NP, D), jnp.float32),
    )(agg, y, dis, b2d)


def _prep_edges(edge_index, g):
    src = edge_index[0].astype(jnp.int32)
    dst = edge_index[1].astype(jnp.int32)
    padlen = EP - E
    src = jnp.concatenate(
        [src + g * NP, jnp.full((padlen,), g * NP + N, jnp.int32)])
    dst = jnp.concatenate([dst, jnp.full((padlen,), N, jnp.int32)])
    return src.reshape(NT, KCH, CH), dst.reshape(NT, KCH, CH)


def kernel(x1, edge_index1, x2, edge_index2, W, b):
    s1, d1 = _prep_edges(edge_index1, 0)
    s2, d2 = _prep_edges(edge_index2, 1)
    src = jnp.stack([s1, s2])
    dst = jnp.stack([d1, d2])
    zpad = jnp.zeros((NP - N, D), jnp.float32)
    xcat = jnp.concatenate([x1, zpad, x2, zpad])

    ones_ch = jnp.ones((CH,), jnp.float32)
    zeros_r = jnp.zeros((RPT,), jnp.float32)
    zeros_rd = jnp.zeros((RPT, D), jnp.float32)

    deg = _deg_kernel(dst, ones_ch, zeros_r)
    y, dis = _prep_call(xcat, deg.reshape(NSC * NP, 1), W)
    agg = _agg_kernel(y, src, dst, zeros_rd)
    out = _finish_call(agg, y, dis, b.reshape(1, D))
    return out[:N], out[NP:NP + N]
